# baseline (speedup 1.0000x reference)
import functools

import jax
import jax.numpy as jnp
from jax import lax
from jax.experimental import pallas as pl
from jax.experimental.pallas import tpu as pltpu
from jax.experimental.pallas import tpu_sc as plsc

_NC = 2
_NS = 16
_NW = _NC * _NS

_B = 4096
_S = 200
_D = 32
_BPW = _B // _NW
_K = 4
_NBLK = _BPW // _K


def _gather_body(idx_hbm, table_hbm, out_hbm, idx_v, rows_v, sem, wsem):
    wid = lax.axis_index("s") * _NC + lax.axis_index("c")
    bbase = wid * _BPW

    pltpu.sync_copy(idx_hbm.at[pl.ds(bbase, _BPW)], idx_v)

    def blk_body(blk, carry):
        slot = lax.rem(blk, 2)
        rows = rows_v.at[slot]

        @pl.when(blk >= 2)
        def _():
            pltpu.make_async_copy(
                out_hbm.at[pl.ds(bbase, _K)], rows, wsem
            ).wait()

        waits = []
        for k in range(_K):
            waits.append(
                pltpu.async_copy(
                    table_hbm.at[idx_v.at[blk * _K + k]],
                    rows.at[k],
                    sem,
                )
            )
        for w in waits:
            w.wait()
        pltpu.async_copy(rows, out_hbm.at[pl.ds(bbase + blk * _K, _K)], wsem)
        return carry

    lax.fori_loop(0, _NBLK, blk_body, 0)

    for slot in range(2):
        pltpu.make_async_copy(
            out_hbm.at[pl.ds(bbase, _K)], rows_v.at[slot], wsem
        ).wait()


@functools.partial(
    pl.kernel,
    out_type=jax.ShapeDtypeStruct((_B, _S, _D), jnp.float32),
    mesh=plsc.VectorSubcoreMesh(core_axis_name="c", subcore_axis_name="s"),
    scratch_types=[
        pltpu.VMEM((_BPW, _S), jnp.int32),
        pltpu.VMEM((2, _K, _S, _D), jnp.float32),
        pltpu.SemaphoreType.DMA,
        pltpu.SemaphoreType.DMA,
    ],
    compiler_params=pltpu.CompilerParams(use_tc_tiling_on_sc=False),
)
def _gather_call(idx_hbm, table_hbm, out_hbm, idx_v, rows_v, sem, wsem):
    _gather_body(idx_hbm, table_hbm, out_hbm, idx_v, rows_v, sem, wsem)


@jax.jit
def kernel(token_ids, embedding_table):
    return _gather_call(token_ids.astype(jnp.int32), embedding_table)
